# Initial kernel scaffold; baseline (speedup 1.0000x reference)
import functools

import jax
import jax.numpy as jnp
from jax import lax
from jax.experimental import pallas as pl
from jax.experimental.pallas import tpu as pltpu
from jax.experimental.pallas import tpu_sc as plsc

_F = 26
_V = 100000
_D = 32
_B = 4096
_H = 50
_N = _B * _H * _F

_NC, _NS, _L = 2, 16, 16
_NW = _NC * _NS
_NPW = _N // _NW
_CH = 1664
_NCHUNK = _NPW // _CH
_GG = 128
_NG = _CH // _GG


def _make_kernel():
    mesh = plsc.VectorSubcoreMesh(core_axis_name="c", subcore_axis_name="s")

    @functools.partial(
        pl.kernel,
        mesh=mesh,
        out_type=jax.ShapeDtypeStruct((_N, _D), jnp.float32),
        scratch_types=[
            pltpu.VMEM((_CH,), jnp.int32),
            pltpu.VMEM((_CH,), jnp.int32),
            pltpu.VMEM((_CH, _D), jnp.float32),
            pltpu.SemaphoreType.DMA,
        ],
    )
    def k(x_hbm, tab_hbm, out_hbm, offs_v, idx_v, rows_v, sem):
        wid = lax.axis_index("s") * _NC + lax.axis_index("c")
        base = wid * _NPW

        for i in range(_CH // _L):
            p = lax.iota(jnp.int32, _L) + (i * _L)
            offs_v[pl.ds(i * _L, _L)] = lax.rem(p, _F) * _V

        def chunk(c, carry):
            start = base + c * _CH
            pltpu.sync_copy(x_hbm.at[pl.ds(start, _CH)], idx_v)
            for i in range(_CH // _L):
                s = pl.ds(i * _L, _L)
                idx_v[s] = idx_v[s] + offs_v[s]
            cps = []
            for j in range(_NG):
                g = pl.ds(j * _GG, _GG)
                cps.append(
                    pltpu.async_copy(tab_hbm.at[idx_v.at[g]], rows_v.at[g], sem)
                )
            for cp in cps:
                cp.wait()
            pltpu.sync_copy(rows_v, out_hbm.at[pl.ds(start, _CH)])
            return carry

        lax.fori_loop(0, _NCHUNK, chunk, 0)

    return k


_gather_kernel = _make_kernel()


@jax.jit
def kernel(x, tables):
    xf = x.reshape(-1).astype(jnp.int32)
    tf = tables.reshape(_F * _V, _D)
    out = _gather_kernel(xf, tf)
    return out.reshape(_B, _H, _F * _D)


# speedup vs baseline: 7.9559x; 7.9559x over previous
import functools

import jax
import jax.numpy as jnp
from jax import lax
from jax.experimental import pallas as pl
from jax.experimental.pallas import tpu as pltpu
from jax.experimental.pallas import tpu_sc as plsc

_F = 26
_V = 100000
_D = 32
_B = 4096
_H = 50
_N = _B * _H * _F

_NC, _NS, _L = 2, 16, 16
_NW = _NC * _NS
_NPW = _N // _NW
_CH = 1664
_NCHUNK = _NPW // _CH
_GG = 128
_NG = _CH // _GG


def _make_kernel():
    mesh = plsc.VectorSubcoreMesh(core_axis_name="c", subcore_axis_name="s")

    @functools.partial(
        pl.kernel,
        mesh=mesh,
        out_type=jax.ShapeDtypeStruct((_N, _D), jnp.float32),
        compiler_params=pltpu.CompilerParams(use_tc_tiling_on_sc=False),
        scratch_types=[
            pltpu.VMEM((_CH,), jnp.int32),
            pltpu.VMEM((_CH,), jnp.int32),
            pltpu.VMEM((_CH, _D), jnp.float32),
            pltpu.SemaphoreType.DMA,
        ],
    )
    def k(x_hbm, tab_hbm, out_hbm, offs_v, idx_v, rows_v, sem):
        wid = lax.axis_index("s") * _NC + lax.axis_index("c")
        base = wid * _NPW

        for i in range(_CH // _L):
            p = lax.iota(jnp.int32, _L) + (i * _L)
            offs_v[pl.ds(i * _L, _L)] = lax.rem(p, _F) * _V

        def chunk(c, carry):
            start = base + c * _CH
            pltpu.sync_copy(x_hbm.at[pl.ds(start, _CH)], idx_v)
            for i in range(_CH // _L):
                s = pl.ds(i * _L, _L)
                idx_v[s] = idx_v[s] + offs_v[s]
            cps = []
            for j in range(_NG):
                g = pl.ds(j * _GG, _GG)
                cps.append(
                    pltpu.async_copy(tab_hbm.at[idx_v.at[g]], rows_v.at[g], sem)
                )
            for cp in cps:
                cp.wait()
            pltpu.sync_copy(rows_v, out_hbm.at[pl.ds(start, _CH)])
            return carry

        lax.fori_loop(0, _NCHUNK, chunk, 0)

    return k


_gather_kernel = _make_kernel()


@jax.jit
def kernel(x, tables):
    xf = x.reshape(-1).astype(jnp.int32)
    tf = tables.reshape(_F * _V, _D)
    out = _gather_kernel(xf, tf)
    return out.reshape(_B, _H, _F * _D)
